# Initial kernel scaffold; baseline (speedup 1.0000x reference)
import jax
import jax.numpy as jnp
from jax.experimental import pallas as pl


def kernel(x_nchw, w_oihw, bias):
    raise NotImplementedError("write your pallas kernel here")


# speedup vs baseline: 1.0410x; 1.0410x over previous
import functools

import jax
import jax.numpy as jnp
from jax import lax
from jax.experimental import pallas as pl
from jax.experimental.pallas import tpu as pltpu


def _ceil_to(x, m):
    return (x + m - 1) // m * m


def _conv_stats_pool_kernel(cols_ref, w_ref, y_ref, pool_ref, stats_ref, *,
                            c_in, tm, ho, wo, mask_bot, mask_right):
    cols = cols_ref[...]

    y = jnp.dot(cols, w_ref[...], preferred_element_type=jnp.float32)
    y_ref[...] = y.astype(jnp.bfloat16)
    stats_ref[0:1, :] = jnp.sum(y, axis=0, keepdims=True)
    stats_ref[1:2, :] = jnp.sum(y * y, axis=0, keepdims=True)

    ridx = pl.program_id(0) * tm + lax.broadcasted_iota(jnp.int32, (tm, 1), 0)
    ow = lax.rem(ridx, wo)
    oh = lax.rem(lax.div(ridx, wo), ho)
    top = oh == 0
    left = ow == 0
    bot = (oh == ho - 1) if mask_bot else None
    right = (ow == wo - 1) if mask_right else None
    neg = jnp.bfloat16(-1e30)

    pool = cols[:, 4 * c_in:5 * c_in]
    for t in (0, 1, 2, 3, 5, 6, 7, 8):
        kh, kw = divmod(t, 3)
        v = cols[:, t * c_in:(t + 1) * c_in]
        bad = None
        if kh == 0:
            bad = top
        elif kh == 2 and bot is not None:
            bad = bot
        if kw == 0:
            bad = left if bad is None else (bad | left)
        elif kw == 2 and right is not None:
            bad = right if bad is None else (bad | right)
        if bad is not None:
            v = jnp.where(bad, neg, v)
        pool = jnp.maximum(pool, v)
    pool_ref[...] = pool


def _apply_kernel(y_ref, pool_ref, bnp_ref, out_ref, *, ch3, c_in):
    y = y_ref[...].astype(jnp.float32)
    out_ref[:, 0:ch3] = jnp.maximum(y * bnp_ref[0:1, :] + bnp_ref[1:2, :], 0.0)
    out_ref[:, ch3:ch3 + c_in] = pool_ref[...].astype(jnp.float32)


def kernel(x_nchw, w_oihw, bias, *, eps=1e-5):
    del bias

    N, C_in, H, W = x_nchw.shape
    Ch3 = w_oihw.shape[0]
    Ho = (H + 2 - 3) // 2 + 1
    Wo = (W + 2 - 3) // 2 + 1
    M = N * Ho * Wo
    K = 9 * C_in
    Cout = Ch3 + C_in
    Cp = _ceil_to(Cout, 128)

    TM = min(1024, _ceil_to(max(M, 8), 8))
    M_pad = _ceil_to(M, TM)
    n_tiles = M_pad // TM

    x = jnp.transpose(x_nchw, (0, 2, 3, 1))
    xp = jnp.pad(x, ((0, 0), (1, 1), (1, 1), (0, 0))).astype(jnp.bfloat16)
    patches = [xp[:, kh:kh + 2 * Ho - 1:2, kw:kw + 2 * Wo - 1:2, :]
               for kh in range(3) for kw in range(3)]
    cols = jnp.stack(patches, axis=3).reshape(M, K)
    cols = jnp.pad(cols, ((0, M_pad - M), (0, 0)))

    w_mat = jnp.transpose(w_oihw, (2, 3, 1, 0)).reshape(K, Ch3).astype(jnp.bfloat16)

    mask_bot = 2 * Ho - 1 >= H
    mask_right = 2 * Wo - 1 >= W

    cparams = pltpu.CompilerParams(dimension_semantics=("parallel",),
                                   vmem_limit_bytes=48 * 1024 * 1024)

    cost1 = pl.CostEstimate(
        flops=2 * M_pad * K * Ch3 + 3 * M_pad * Ch3 + 10 * M_pad * C_in,
        transcendentals=0,
        bytes_accessed=2 * (M_pad * K + K * Ch3 + M_pad * (Ch3 + C_in)) + 4 * n_tiles * 8 * Ch3)
    y_bf, pool_bf, stats = pl.pallas_call(
        functools.partial(_conv_stats_pool_kernel, c_in=C_in, tm=TM, ho=Ho, wo=Wo,
                          mask_bot=mask_bot, mask_right=mask_right),
        out_shape=(jax.ShapeDtypeStruct((M_pad, Ch3), jnp.bfloat16),
                   jax.ShapeDtypeStruct((M_pad, C_in), jnp.bfloat16),
                   jax.ShapeDtypeStruct((n_tiles * 8, Ch3), jnp.float32)),
        grid=(n_tiles,),
        in_specs=[pl.BlockSpec((TM, K), lambda i: (i, 0)),
                  pl.BlockSpec((K, Ch3), lambda i: (0, 0))],
        out_specs=(pl.BlockSpec((TM, Ch3), lambda i: (i, 0)),
                   pl.BlockSpec((TM, C_in), lambda i: (i, 0)),
                   pl.BlockSpec((8, Ch3), lambda i: (i, 0))),
        compiler_params=cparams,
        cost_estimate=cost1,
    )(cols, w_mat)

    part = stats.reshape(n_tiles, 8, Ch3)
    mean = jnp.sum(part[:, 0, :], axis=0) / M
    var = jnp.maximum(jnp.sum(part[:, 1, :], axis=0) / M - mean * mean, 0.0)
    inv_std = lax.rsqrt(var + eps)
    bnp = jnp.zeros((8, Ch3), jnp.float32).at[0].set(inv_std).at[1].set(-mean * inv_std)

    cost2 = pl.CostEstimate(
        flops=3 * M_pad * Ch3,
        transcendentals=0,
        bytes_accessed=2 * M_pad * (Ch3 + C_in) + 4 * M_pad * Cp)
    fused = pl.pallas_call(
        functools.partial(_apply_kernel, ch3=Ch3, c_in=C_in),
        out_shape=jax.ShapeDtypeStruct((M_pad, Cp), jnp.float32),
        grid=(n_tiles,),
        in_specs=[pl.BlockSpec((TM, Ch3), lambda i: (i, 0)),
                  pl.BlockSpec((TM, C_in), lambda i: (i, 0)),
                  pl.BlockSpec((8, Ch3), lambda i: (0, 0))],
        out_specs=pl.BlockSpec((TM, Cp), lambda i: (i, 0)),
        compiler_params=cparams,
        cost_estimate=cost2,
    )(y_bf, pool_bf, bnp)

    out = fused[:M, :Cout].reshape(N, Ho, Wo, Cout)
    return jnp.transpose(out, (0, 3, 1, 2))


# speedup vs baseline: 25.9598x; 24.9383x over previous
import functools

import jax
import jax.numpy as jnp
from jax import lax
from jax.experimental import pallas as pl
from jax.experimental.pallas import tpu as pltpu


def _ceil_to(x, m):
    return (x + m - 1) // m * m


def _conv_pool_stats_kernel(x_ref, w_ref, y_ref, pool_ref, stats_ref, *,
                            c_in, ch3, ho, wo, wop, jp,
                            mask_bot, mask_right):
    xb = x_ref[0]
    mrows = ho * wop
    xe = xb[:, 0]
    xo = xb[:, 1]

    rows = [xe[0:ho], xo[0:ho], xe[1:ho + 1]]

    y = jnp.zeros((mrows, ch3), jnp.float32)
    for kh in range(3):
        p0 = rows[kh][:, 0:wop, :].reshape(mrows, 2 * c_in)
        p1 = rows[kh][:, 1:wop + 1, :].reshape(mrows, 2 * c_in)
        y = y + jnp.dot(p0, w_ref[kh * 4 * c_in:kh * 4 * c_in + 2 * c_in],
                        preferred_element_type=jnp.float32)
        y = y + jnp.dot(p1, w_ref[kh * 4 * c_in + 2 * c_in:(kh + 1) * 4 * c_in],
                        preferred_element_type=jnp.float32)
    y_ref[0] = y.astype(jnp.bfloat16)

    owp_idx = lax.rem(lax.broadcasted_iota(jnp.int32, (mrows, 1), 0),
                      jnp.int32(wop))
    valid = owp_idx < wo
    ys = jnp.where(valid, y, 0.0)
    stats_ref[0, 0:1, :] = jnp.sum(ys, axis=0, keepdims=True)
    stats_ref[0, 1:2, :] = jnp.sum(ys * ys, axis=0, keepdims=True)

    neg = jnp.bfloat16(-1e30)
    d0 = lax.broadcasted_iota(jnp.int32, (ho, jp, 2 * c_in), 0)
    t0 = jnp.where(d0 == 0, neg, rows[0])
    t2 = jnp.where(d0 == ho - 1, neg, rows[2]) if mask_bot else rows[2]
    m1 = jnp.maximum(jnp.maximum(t0, rows[1]), t2)
    d1 = lax.broadcasted_iota(jnp.int32, (ho, jp, 2 * c_in), 1)
    dl = lax.broadcasted_iota(jnp.int32, (ho, jp, 2 * c_in), 2)
    m1 = jnp.where((d1 == 0) & (dl < c_in), neg, m1)
    if mask_right:
        m1 = jnp.where((d1 == wo) & (dl < c_in), neg, m1)
    pa = m1[:, 0:wop, :].reshape(mrows, 2 * c_in)
    pb = m1[:, 1:wop + 1, :].reshape(mrows, 2 * c_in)
    pool = jnp.maximum(jnp.maximum(pa[:, 0:c_in], pa[:, c_in:2 * c_in]),
                       pb[:, 0:c_in])
    pool_ref[0] = pool


def _apply_kernel(y_ref, pool_ref, bnp_ref, out_ref, *, ch3, c_in):
    y = y_ref[...].astype(jnp.float32)
    out_ref[:, 0:ch3] = jnp.maximum(y * bnp_ref[0:1, :] + bnp_ref[1:2, :], 0.0)
    out_ref[:, ch3:ch3 + c_in] = pool_ref[...].astype(jnp.float32)


def kernel(x_nchw, w_oihw, bias, *, eps=1e-5):
    del bias

    N, C_in, H, W = x_nchw.shape
    Ch3 = w_oihw.shape[0]
    Ho = (H + 2 - 3) // 2 + 1
    Wo = (W + 2 - 3) // 2 + 1
    Wop = _ceil_to(Wo, 8)
    Jp = Wop + 1
    Hp = 2 * (Ho + 1)
    M = N * Ho * Wo
    Mr = Ho * Wop
    Cout = Ch3 + C_in

    x = jnp.transpose(x_nchw, (0, 2, 3, 1)).astype(jnp.bfloat16)
    xp = jnp.pad(x, ((0, 0), (1, Hp - 1 - H), (1, 2 * Jp - 1 - W), (0, 0)))
    xpp = xp.reshape(N, Hp // 2, 2, Jp, 2 * C_in)

    wt = jnp.transpose(w_oihw, (2, 3, 1, 0)).astype(jnp.bfloat16)
    blocks = []
    for kh in range(3):
        blocks.append(wt[kh, 0])
        blocks.append(wt[kh, 1])
        blocks.append(wt[kh, 2])
        blocks.append(jnp.zeros((C_in, Ch3), jnp.bfloat16))
    wcat = jnp.concatenate(blocks, axis=0)

    mask_bot = 2 * Ho - 1 >= H
    mask_right = 2 * Wo - 1 >= W

    cparams = pltpu.CompilerParams(dimension_semantics=("parallel",),
                                   vmem_limit_bytes=48 * 1024 * 1024)

    cost1 = pl.CostEstimate(
        flops=2 * N * Mr * 6 * 2 * C_in * Ch3 + 12 * N * Mr * C_in,
        transcendentals=0,
        bytes_accessed=2 * N * (Hp * Jp * 2 * C_in + Mr * (Ch3 + C_in)) + 4 * N * 8 * Ch3)
    y_bf, pool_bf, stats = pl.pallas_call(
        functools.partial(_conv_pool_stats_kernel, c_in=C_in, ch3=Ch3,
                          ho=Ho, wo=Wo, wop=Wop, jp=Jp,
                          mask_bot=mask_bot, mask_right=mask_right),
        out_shape=(jax.ShapeDtypeStruct((N, Mr, Ch3), jnp.bfloat16),
                   jax.ShapeDtypeStruct((N, Mr, C_in), jnp.bfloat16),
                   jax.ShapeDtypeStruct((N, 8, Ch3), jnp.float32)),
        grid=(N,),
        in_specs=[pl.BlockSpec((1, Hp // 2, 2, Jp, 2 * C_in), lambda i: (i, 0, 0, 0, 0)),
                  pl.BlockSpec((12 * C_in, Ch3), lambda i: (0, 0))],
        out_specs=(pl.BlockSpec((1, Mr, Ch3), lambda i: (i, 0, 0)),
                   pl.BlockSpec((1, Mr, C_in), lambda i: (i, 0, 0)),
                   pl.BlockSpec((1, 8, Ch3), lambda i: (i, 0, 0))),
        compiler_params=cparams,
        cost_estimate=cost1,
    )(xpp, wcat)

    mean = jnp.sum(stats[:, 0, :], axis=0) / M
    var = jnp.maximum(jnp.sum(stats[:, 1, :], axis=0) / M - mean * mean, 0.0)
    inv_std = lax.rsqrt(var + eps)
    bnp = jnp.zeros((8, Ch3), jnp.float32).at[0].set(inv_std).at[1].set(-mean * inv_std)

    M2 = N * Mr
    TM = min(1024, M2)
    n2 = _ceil_to(M2, TM) // TM
    cost2 = pl.CostEstimate(
        flops=3 * M2 * Ch3,
        transcendentals=0,
        bytes_accessed=2 * M2 * (Ch3 + C_in) + 4 * M2 * (Ch3 + C_in))
    fused = pl.pallas_call(
        functools.partial(_apply_kernel, ch3=Ch3, c_in=C_in),
        out_shape=jax.ShapeDtypeStruct((M2, Cout), jnp.float32),
        grid=(n2,),
        in_specs=[pl.BlockSpec((TM, Ch3), lambda i: (i, 0)),
                  pl.BlockSpec((TM, C_in), lambda i: (i, 0)),
                  pl.BlockSpec((8, Ch3), lambda i: (0, 0))],
        out_specs=pl.BlockSpec((TM, Cout), lambda i: (i, 0)),
        compiler_params=cparams,
        cost_estimate=cost2,
    )(y_bf.reshape(M2, Ch3), pool_bf.reshape(M2, C_in), bnp)

    out = fused.reshape(N, Ho, Wop, Cout)[:, :, :Wo, :]
    return jnp.transpose(out, (0, 3, 1, 2))
